# baseline (speedup 1.0000x reference)
import functools

import jax
import jax.numpy as jnp
from jax import lax
from jax.experimental import pallas as pl
from jax.experimental.pallas import tpu as pltpu
from jax.experimental.pallas import tpu_sc as plsc

T = 2048
D = 1024
E = 8
FF = 1024
SF = 1024
TK = 2 * T
BLK = 128
NBP = TK // BLK + E - 1
NSTEPS = NBP
TKP = NBP * BLK
BT2 = 256

_F32 = jnp.float32
_BF16 = jnp.bfloat16
_I32 = jnp.int32


def _router_body(x_ref, gw_ref, bias_ref, pos_ref, p0_ref, p1_ref,
                 w1_ref, w2_ref, meta_ref):
    x = x_ref[...]
    logits = lax.dot_general(
        x, gw_ref[...], (((1,), (1,)), ((), ())),
        preferred_element_type=_F32,
        precision=lax.Precision.DEFAULT,
    )
    scores = jax.nn.sigmoid(logits)
    sfc = scores + bias_ref[...]

    lane = lax.broadcasted_iota(_I32, (T, E), 1)
    big = _F32(1e30)

    m1 = jnp.max(sfc, axis=1, keepdims=True)
    i1 = jnp.min(jnp.where(sfc >= m1, lane, E), axis=1, keepdims=True)
    oh1 = lane == i1
    sfc2 = jnp.where(oh1, -big, sfc)
    m2 = jnp.max(sfc2, axis=1, keepdims=True)
    i2 = jnp.min(jnp.where(sfc2 >= m2, lane, E), axis=1, keepdims=True)
    oh2 = lane == i2

    w1 = jnp.sum(jnp.where(oh1, scores, 0.0), axis=1, keepdims=True)
    w2 = jnp.sum(jnp.where(oh2, scores, 0.0), axis=1, keepdims=True)
    norm = w1 + w2 + 1e-20
    w1 = w1 / norm
    w2 = w2 / norm

    oh_f = oh1.astype(_F32) + oh2.astype(_F32)
    oh_b = oh_f.astype(_BF16)

    CH = 128
    tri = (lax.broadcasted_iota(_I32, (CH, CH), 0)
           > lax.broadcasted_iota(_I32, (CH, CH), 1)).astype(_BF16)
    chunks = []
    running = jnp.zeros((1, E), _F32)
    for c in range(T // CH):
        blk = oh_b[c * CH:(c + 1) * CH, :]
        within = lax.dot_general(
            tri, blk, (((1,), (0,)), ((), ())), preferred_element_type=_F32)
        chunks.append(within + running)
        running = running + jnp.sum(blk.astype(_F32), axis=0, keepdims=True)
    cexcl = jnp.concatenate(chunks, axis=0)
    counts = running

    cb = jnp.floor((counts + (BLK - 1.0)) * (1.0 / BLK))
    cb_b = jnp.broadcast_to(cb * BLK, (T, E))
    off1 = jnp.sum(jnp.where(lane < i1, cb_b, 0.0), axis=1, keepdims=True)
    off2 = jnp.sum(jnp.where(lane < i2, cb_b, 0.0), axis=1, keepdims=True)
    rank1 = jnp.sum(jnp.where(oh1, cexcl, 0.0), axis=1, keepdims=True)
    rank2 = jnp.sum(jnp.where(oh2, cexcl, 0.0), axis=1, keepdims=True)
    pos1 = (off1 + rank1).astype(_I32)
    pos2 = (off2 + rank2).astype(_I32)

    pos_ref[...] = jnp.concatenate([pos1, pos2], axis=1)
    p0_ref[...] = pos1
    p1_ref[...] = pos2
    w1_ref[...] = w1
    w2_ref[...] = w2

    ones_col = jnp.ones((T, 1), _BF16)
    counts_col = lax.dot_general(
        oh_b, ones_col, (((0,), (0,)), ((), ())),
        preferred_element_type=_F32)
    ltri = (lax.broadcasted_iota(_I32, (E, E), 1)
            < lax.broadcasted_iota(_I32, (E, E), 0)).astype(_BF16)
    cb_col = jnp.floor((counts_col + (BLK - 1.0)) * (1.0 / BLK))
    start = lax.dot_general(
        ltri, cb_col.astype(_BF16), (((1,), (0,)), ((), ())),
        preferred_element_type=_F32).astype(_I32)
    total = jnp.sum(cb_col).astype(_I32)

    svec = lax.broadcasted_iota(_I32, (E, 128), 1)
    s_eff = jnp.minimum(svec, total - 1)
    start_b2 = jnp.broadcast_to(start, (E, 128))
    e_of_s = jnp.sum((start_b2 <= s_eff).astype(_I32), axis=0,
                     keepdims=True) - 1
    blk_s = s_eff[0:1, :]
    valid = (svec[0:1, :] < total).astype(_I32)

    meta_ref[...] = jnp.concatenate(
        [e_of_s, blk_s, valid, jnp.zeros((5, 128), _I32)], axis=0)


def _run_router(x, gate_w, bias2):
    return pl.pallas_call(
        _router_body,
        out_shape=(
            jax.ShapeDtypeStruct((T, 2), _I32),
            jax.ShapeDtypeStruct((T, 1), _I32),
            jax.ShapeDtypeStruct((T, 1), _I32),
            jax.ShapeDtypeStruct((T, 1), _F32),
            jax.ShapeDtypeStruct((T, 1), _F32),
            jax.ShapeDtypeStruct((8, 128), _I32),
        ),
    )(x, gate_w, bias2)


def _make_sc_dispatch():
    mesh = plsc.VectorSubcoreMesh(core_axis_name="c", subcore_axis_name="s")
    NW = 32
    CHUNK = TK // NW
    SUB = CHUNK // 2

    @functools.partial(
        pl.kernel, mesh=mesh,
        out_type=jax.ShapeDtypeStruct((TKP, D), _F32),
        scratch_types=[
            pltpu.VMEM((SUB,), _I32),
            pltpu.VMEM((SUB,), _I32),
            pltpu.VMEM((SUB, D), _F32),
            pltpu.SemaphoreType.DMA,
        ],
    )
    def sc_dispatch(x_hbm, pos_hbm, xs_hbm, pos_v, tok_v, rows_v, sem):
        wid = lax.axis_index("s") * 2 + lax.axis_index("c")
        base = wid * CHUNK
        for sub in range(2):
            sbase = base + sub * SUB
            pltpu.sync_copy(pos_hbm.at[pl.ds(sbase, SUB)], pos_v)
            for j in range(SUB // 16):
                t16 = lax.shift_right_logical(
                    lax.iota(_I32, 16), 1) + ((sbase + 16 * j) // 2)
                tok_v[pl.ds(16 * j, 16)] = t16
            pltpu.async_copy(x_hbm.at[tok_v], rows_v, sem).wait()
            pltpu.async_copy(rows_v, xs_hbm.at[pos_v], sem).wait()

    return sc_dispatch


_SC_CACHE = {}


def _get_sc_dispatch():
    if "dispatch" not in _SC_CACHE:
        _SC_CACHE["dispatch"] = _make_sc_dispatch()
    return _SC_CACHE["dispatch"]


def _grouped_body(meta_ref, xs_ref, wgu_ref, wdn_ref, rs_ref):
    s = pl.program_id(0)
    valid = meta_ref[2, s] == 1

    @pl.when(valid)
    def _():
        xb = xs_ref[...].astype(_BF16)
        wgu = wgu_ref[0].astype(_BF16)
        gu = lax.dot_general(xb, wgu, (((1,), (1,)), ((), ())),
                             preferred_element_type=_F32)
        g = gu[:, :FF]
        u = gu[:, FF:]
        h = (g * jax.nn.sigmoid(g) * u).astype(_BF16)
        wdn = wdn_ref[0].astype(_BF16)
        eo = lax.dot_general(h, wdn, (((1,), (1,)), ((), ())),
                             preferred_element_type=_F32)
        rs_ref[...] = eo


def _run_grouped(meta, xs, w_gate_up, w_down):
    grid_spec = pltpu.PrefetchScalarGridSpec(
        num_scalar_prefetch=1,
        grid=(NSTEPS,),
        in_specs=[
            pl.BlockSpec((BLK, D), lambda s, m: (m[1, s], 0)),
            pl.BlockSpec((1, 2 * FF, D), lambda s, m: (m[0, s], 0, 0)),
            pl.BlockSpec((1, D, FF), lambda s, m: (m[0, s], 0, 0)),
        ],
        out_specs=pl.BlockSpec((BLK, D), lambda s, m: (m[1, s], 0)),
    )
    return pl.pallas_call(
        _grouped_body,
        grid_spec=grid_spec,
        out_shape=jax.ShapeDtypeStruct((TKP, D), _F32),
    )(meta, xs, w_gate_up, w_down)


def _make_sc_gather():
    mesh = plsc.VectorSubcoreMesh(core_axis_name="c", subcore_axis_name="s")
    NW = 32
    TPW = T // NW

    @functools.partial(
        pl.kernel, mesh=mesh,
        out_type=(
            jax.ShapeDtypeStruct((T, D), _F32),
            jax.ShapeDtypeStruct((T, D), _F32),
        ),
        scratch_types=[
            pltpu.VMEM((TPW,), _I32),
            pltpu.VMEM((TPW, D), _F32),
            pltpu.SemaphoreType.DMA,
        ],
    )
    def sc_gather(rs_hbm, p0_hbm, p1_hbm, a_hbm, b_hbm, idx_v, rows_v, sem):
        wid = lax.axis_index("s") * 2 + lax.axis_index("c")
        base = wid * TPW
        pltpu.sync_copy(p0_hbm.at[pl.ds(base, TPW)], idx_v)
        pltpu.async_copy(rs_hbm.at[idx_v], rows_v, sem).wait()
        pltpu.sync_copy(rows_v, a_hbm.at[pl.ds(base, TPW)])
        pltpu.sync_copy(p1_hbm.at[pl.ds(base, TPW)], idx_v)
        pltpu.async_copy(rs_hbm.at[idx_v], rows_v, sem).wait()
        pltpu.sync_copy(rows_v, b_hbm.at[pl.ds(base, TPW)])

    return sc_gather


def _get_sc_gather():
    if "gather" not in _SC_CACHE:
        _SC_CACHE["gather"] = _make_sc_gather()
    return _SC_CACHE["gather"]


def _shared_body(x_ref, sgu_ref, sdn_ref, out_ref, sgu_c, sdn_c):
    @pl.when(pl.program_id(0) == 0)
    def _():
        sgu_c[...] = sgu_ref[...].astype(_BF16)
        sdn_c[...] = sdn_ref[...].astype(_BF16)

    xb = x_ref[...].astype(_BF16)
    sgu = lax.dot_general(xb, sgu_c[...], (((1,), (1,)), ((), ())),
                          preferred_element_type=_F32)
    sg = sgu[:, :SF]
    su = sgu[:, SF:]
    sh = (sg * jax.nn.sigmoid(sg) * su).astype(_BF16)
    out_ref[...] = lax.dot_general(sh, sdn_c[...], (((1,), (1,)), ((), ())),
                                   preferred_element_type=_F32)


def _run_shared(x, shared_gate_up, shared_down):
    return pl.pallas_call(
        _shared_body,
        grid=(T // BT2,),
        in_specs=[
            pl.BlockSpec((BT2, D), lambda i: (i, 0)),
            pl.BlockSpec((2 * SF, D), lambda i: (0, 0)),
            pl.BlockSpec((D, SF), lambda i: (0, 0)),
        ],
        out_specs=pl.BlockSpec((BT2, D), lambda i: (i, 0)),
        out_shape=jax.ShapeDtypeStruct((T, D), _F32),
        scratch_shapes=[
            pltpu.VMEM((2 * SF, D), _BF16),
            pltpu.VMEM((D, SF), _BF16),
        ],
    )(x, shared_gate_up, shared_down)


def _combine_body(sh_ref, a_ref, b_ref, w1_ref, w2_ref, out_ref):
    out_ref[...] = (sh_ref[...] + w1_ref[...] * a_ref[...]
                    + w2_ref[...] * b_ref[...])


def _run_combine(sh, a, b, w1, w2):
    return pl.pallas_call(
        _combine_body,
        grid=(T // BT2,),
        in_specs=[
            pl.BlockSpec((BT2, D), lambda i: (i, 0)),
            pl.BlockSpec((BT2, D), lambda i: (i, 0)),
            pl.BlockSpec((BT2, D), lambda i: (i, 0)),
            pl.BlockSpec((BT2, 1), lambda i: (i, 0)),
            pl.BlockSpec((BT2, 1), lambda i: (i, 0)),
        ],
        out_specs=pl.BlockSpec((BT2, D), lambda i: (i, 0)),
        out_shape=jax.ShapeDtypeStruct((T, D), _F32),
    )(sh, a, b, w1, w2)


def kernel(hidden_states, gate_w, expert_bias, w_gate_up, w_down,
           shared_gate_up, shared_down):
    orig_shape = hidden_states.shape
    x = hidden_states.reshape(-1, orig_shape[-1])
    bias2 = expert_bias.reshape(1, E)

    pos2, p0, p1, w1, w2, meta = _run_router(x, gate_w, bias2)
    pos_flat = pos2.reshape(TK)

    xs = _get_sc_dispatch()(x, pos_flat)

    rs = _run_grouped(meta, xs, w_gate_up, w_down)

    sh = _run_shared(x, shared_gate_up, shared_down)
    a, b = _get_sc_gather()(rs, p0.reshape(T), p1.reshape(T))

    out = _run_combine(sh, a, b, w1, w2)
    return out.reshape(orig_shape)


# speedup vs baseline: 1.3373x; 1.3373x over previous
import functools

import jax
import jax.numpy as jnp
from jax import lax
from jax.experimental import pallas as pl
from jax.experimental.pallas import tpu as pltpu
from jax.experimental.pallas import tpu_sc as plsc

T = 2048
D = 1024
E = 8
FF = 1024
SF = 1024
TK = 2 * T
BLK = 512
NBP = TK // BLK + E - 1
NSTEPS = NBP
TKP = NBP * BLK
BT2 = 256

_F32 = jnp.float32
_BF16 = jnp.bfloat16
_I32 = jnp.int32


def _router_body(x_ref, gw_ref, bias_ref, pos_ref, p0_ref, p1_ref,
                 w1_ref, w2_ref, meta_ref):
    x = x_ref[...]
    logits = lax.dot_general(
        x, gw_ref[...], (((1,), (1,)), ((), ())),
        preferred_element_type=_F32,
        precision=lax.Precision.DEFAULT,
    )
    scores = jax.nn.sigmoid(logits)
    sfc = scores + bias_ref[...]

    lane = lax.broadcasted_iota(_I32, (T, E), 1)
    big = _F32(1e30)

    m1 = jnp.max(sfc, axis=1, keepdims=True)
    i1 = jnp.min(jnp.where(sfc >= m1, lane, E), axis=1, keepdims=True)
    oh1 = lane == i1
    sfc2 = jnp.where(oh1, -big, sfc)
    m2 = jnp.max(sfc2, axis=1, keepdims=True)
    i2 = jnp.min(jnp.where(sfc2 >= m2, lane, E), axis=1, keepdims=True)
    oh2 = lane == i2

    w1 = jnp.sum(jnp.where(oh1, scores, 0.0), axis=1, keepdims=True)
    w2 = jnp.sum(jnp.where(oh2, scores, 0.0), axis=1, keepdims=True)
    norm = w1 + w2 + 1e-20
    w1 = w1 / norm
    w2 = w2 / norm

    oh_f = oh1.astype(_F32) + oh2.astype(_F32)
    oh_b = oh_f.astype(_BF16)

    CH = 128
    tri = (lax.broadcasted_iota(_I32, (CH, CH), 0)
           > lax.broadcasted_iota(_I32, (CH, CH), 1)).astype(_BF16)
    chunks = []
    running = jnp.zeros((1, E), _F32)
    for c in range(T // CH):
        blk = oh_b[c * CH:(c + 1) * CH, :]
        within = lax.dot_general(
            tri, blk, (((1,), (0,)), ((), ())), preferred_element_type=_F32)
        chunks.append(within + running)
        running = running + jnp.sum(blk.astype(_F32), axis=0, keepdims=True)
    cexcl = jnp.concatenate(chunks, axis=0)
    counts = running

    cb = jnp.floor((counts + (BLK - 1.0)) * (1.0 / BLK))
    cb_b = jnp.broadcast_to(cb * BLK, (T, E))
    off1 = jnp.sum(jnp.where(lane < i1, cb_b, 0.0), axis=1, keepdims=True)
    off2 = jnp.sum(jnp.where(lane < i2, cb_b, 0.0), axis=1, keepdims=True)
    rank1 = jnp.sum(jnp.where(oh1, cexcl, 0.0), axis=1, keepdims=True)
    rank2 = jnp.sum(jnp.where(oh2, cexcl, 0.0), axis=1, keepdims=True)
    pos1 = (off1 + rank1).astype(_I32)
    pos2 = (off2 + rank2).astype(_I32)

    pos_ref[...] = jnp.concatenate([pos1, pos2], axis=1)
    p0_ref[...] = pos1
    p1_ref[...] = pos2
    w1_ref[...] = w1
    w2_ref[...] = w2

    ones_col = jnp.ones((T, 1), _BF16)
    counts_col = lax.dot_general(
        oh_b, ones_col, (((0,), (0,)), ((), ())),
        preferred_element_type=_F32)
    ltri = (lax.broadcasted_iota(_I32, (E, E), 1)
            < lax.broadcasted_iota(_I32, (E, E), 0)).astype(_BF16)
    cb_col = jnp.floor((counts_col + (BLK - 1.0)) * (1.0 / BLK))
    start = lax.dot_general(
        ltri, cb_col.astype(_BF16), (((1,), (0,)), ((), ())),
        preferred_element_type=_F32).astype(_I32)
    total = jnp.sum(cb_col).astype(_I32)

    svec = lax.broadcasted_iota(_I32, (E, 128), 1)
    s_eff = jnp.minimum(svec, total - 1)
    start_b2 = jnp.broadcast_to(start, (E, 128))
    e_of_s = jnp.sum((start_b2 <= s_eff).astype(_I32), axis=0,
                     keepdims=True) - 1
    blk_s = s_eff[0:1, :]
    valid = (svec[0:1, :] < total).astype(_I32)

    meta_ref[...] = jnp.concatenate(
        [e_of_s, blk_s, valid, jnp.zeros((5, 128), _I32)], axis=0)


def _run_router(x, gate_w, bias2):
    return pl.pallas_call(
        _router_body,
        out_shape=(
            jax.ShapeDtypeStruct((T, 2), _I32),
            jax.ShapeDtypeStruct((T, 1), _I32),
            jax.ShapeDtypeStruct((T, 1), _I32),
            jax.ShapeDtypeStruct((T, 1), _F32),
            jax.ShapeDtypeStruct((T, 1), _F32),
            jax.ShapeDtypeStruct((8, 128), _I32),
        ),
    )(x, gate_w, bias2)


def _make_sc_dispatch():
    mesh = plsc.VectorSubcoreMesh(core_axis_name="c", subcore_axis_name="s")
    NW = 32
    CHUNK = TK // NW
    SUB = CHUNK // 2

    @functools.partial(
        pl.kernel, mesh=mesh,
        out_type=jax.ShapeDtypeStruct((TKP, D), _F32),
        scratch_types=[
            pltpu.VMEM((SUB,), _I32),
            pltpu.VMEM((SUB,), _I32),
            pltpu.VMEM((SUB, D), _F32),
            pltpu.SemaphoreType.DMA,
        ],
    )
    def sc_dispatch(x_hbm, pos_hbm, xs_hbm, pos_v, tok_v, rows_v, sem):
        wid = lax.axis_index("s") * 2 + lax.axis_index("c")
        base = wid * CHUNK
        for sub in range(2):
            sbase = base + sub * SUB
            pltpu.sync_copy(pos_hbm.at[pl.ds(sbase, SUB)], pos_v)
            for j in range(SUB // 16):
                t16 = lax.shift_right_logical(
                    lax.iota(_I32, 16), 1) + ((sbase + 16 * j) // 2)
                tok_v[pl.ds(16 * j, 16)] = t16
            pltpu.async_copy(x_hbm.at[tok_v], rows_v, sem).wait()
            pltpu.async_copy(rows_v, xs_hbm.at[pos_v], sem).wait()

    return sc_dispatch


_SC_CACHE = {}


def _get_sc_dispatch():
    if "dispatch" not in _SC_CACHE:
        _SC_CACHE["dispatch"] = _make_sc_dispatch()
    return _SC_CACHE["dispatch"]


def _grouped_body(meta_ref, xs_ref, wgu_ref, wdn_ref, rs_ref):
    s = pl.program_id(0)
    valid = meta_ref[2, s] == 1

    @pl.when(valid)
    def _():
        xb = xs_ref[...].astype(_BF16)
        wgu = wgu_ref[0].astype(_BF16)
        gu = lax.dot_general(xb, wgu, (((1,), (1,)), ((), ())),
                             preferred_element_type=_F32)
        g = gu[:, :FF]
        u = gu[:, FF:]
        h = (g * jax.nn.sigmoid(g) * u).astype(_BF16)
        wdn = wdn_ref[0].astype(_BF16)
        eo = lax.dot_general(h, wdn, (((1,), (1,)), ((), ())),
                             preferred_element_type=_F32)
        rs_ref[...] = eo


def _run_grouped(meta, xs, w_gate_up, w_down):
    grid_spec = pltpu.PrefetchScalarGridSpec(
        num_scalar_prefetch=1,
        grid=(NSTEPS,),
        in_specs=[
            pl.BlockSpec((BLK, D), lambda s, m: (m[1, s], 0)),
            pl.BlockSpec((1, 2 * FF, D), lambda s, m: (m[0, s], 0, 0)),
            pl.BlockSpec((1, D, FF), lambda s, m: (m[0, s], 0, 0)),
        ],
        out_specs=pl.BlockSpec((BLK, D), lambda s, m: (m[1, s], 0)),
    )
    return pl.pallas_call(
        _grouped_body,
        grid_spec=grid_spec,
        out_shape=jax.ShapeDtypeStruct((TKP, D), _F32),
    )(meta, xs, w_gate_up, w_down)


def _make_sc_gather():
    mesh = plsc.VectorSubcoreMesh(core_axis_name="c", subcore_axis_name="s")
    NW = 32
    TPW = T // NW

    @functools.partial(
        pl.kernel, mesh=mesh,
        out_type=(
            jax.ShapeDtypeStruct((T, D), _F32),
            jax.ShapeDtypeStruct((T, D), _F32),
        ),
        scratch_types=[
            pltpu.VMEM((TPW,), _I32),
            pltpu.VMEM((TPW, D), _F32),
            pltpu.SemaphoreType.DMA,
        ],
    )
    def sc_gather(rs_hbm, p0_hbm, p1_hbm, a_hbm, b_hbm, idx_v, rows_v, sem):
        wid = lax.axis_index("s") * 2 + lax.axis_index("c")
        base = wid * TPW
        pltpu.sync_copy(p0_hbm.at[pl.ds(base, TPW)], idx_v)
        pltpu.async_copy(rs_hbm.at[idx_v], rows_v, sem).wait()
        pltpu.sync_copy(rows_v, a_hbm.at[pl.ds(base, TPW)])
        pltpu.sync_copy(p1_hbm.at[pl.ds(base, TPW)], idx_v)
        pltpu.async_copy(rs_hbm.at[idx_v], rows_v, sem).wait()
        pltpu.sync_copy(rows_v, b_hbm.at[pl.ds(base, TPW)])

    return sc_gather


def _get_sc_gather():
    if "gather" not in _SC_CACHE:
        _SC_CACHE["gather"] = _make_sc_gather()
    return _SC_CACHE["gather"]


def _shared_body(x_ref, sgu_ref, sdn_ref, out_ref, sgu_c, sdn_c):
    @pl.when(pl.program_id(0) == 0)
    def _():
        sgu_c[...] = sgu_ref[...].astype(_BF16)
        sdn_c[...] = sdn_ref[...].astype(_BF16)

    xb = x_ref[...].astype(_BF16)
    sgu = lax.dot_general(xb, sgu_c[...], (((1,), (1,)), ((), ())),
                          preferred_element_type=_F32)
    sg = sgu[:, :SF]
    su = sgu[:, SF:]
    sh = (sg * jax.nn.sigmoid(sg) * su).astype(_BF16)
    out_ref[...] = lax.dot_general(sh, sdn_c[...], (((1,), (1,)), ((), ())),
                                   preferred_element_type=_F32)


def _run_shared(x, shared_gate_up, shared_down):
    return pl.pallas_call(
        _shared_body,
        grid=(T // BT2,),
        in_specs=[
            pl.BlockSpec((BT2, D), lambda i: (i, 0)),
            pl.BlockSpec((2 * SF, D), lambda i: (0, 0)),
            pl.BlockSpec((D, SF), lambda i: (0, 0)),
        ],
        out_specs=pl.BlockSpec((BT2, D), lambda i: (i, 0)),
        out_shape=jax.ShapeDtypeStruct((T, D), _F32),
        scratch_shapes=[
            pltpu.VMEM((2 * SF, D), _BF16),
            pltpu.VMEM((D, SF), _BF16),
        ],
    )(x, shared_gate_up, shared_down)


def _combine_body(sh_ref, a_ref, b_ref, w1_ref, w2_ref, out_ref):
    out_ref[...] = (sh_ref[...] + w1_ref[...] * a_ref[...]
                    + w2_ref[...] * b_ref[...])


def _run_combine(sh, a, b, w1, w2):
    return pl.pallas_call(
        _combine_body,
        grid=(T // BT2,),
        in_specs=[
            pl.BlockSpec((BT2, D), lambda i: (i, 0)),
            pl.BlockSpec((BT2, D), lambda i: (i, 0)),
            pl.BlockSpec((BT2, D), lambda i: (i, 0)),
            pl.BlockSpec((BT2, 1), lambda i: (i, 0)),
            pl.BlockSpec((BT2, 1), lambda i: (i, 0)),
        ],
        out_specs=pl.BlockSpec((BT2, D), lambda i: (i, 0)),
        out_shape=jax.ShapeDtypeStruct((T, D), _F32),
    )(sh, a, b, w1, w2)


def kernel(hidden_states, gate_w, expert_bias, w_gate_up, w_down,
           shared_gate_up, shared_down):
    orig_shape = hidden_states.shape
    x = hidden_states.reshape(-1, orig_shape[-1])
    bias2 = expert_bias.reshape(1, E)

    pos2, p0, p1, w1, w2, meta = _run_router(x, gate_w, bias2)
    pos_flat = pos2.reshape(TK)

    xs = _get_sc_dispatch()(x, pos_flat)

    rs = _run_grouped(meta, xs, w_gate_up, w_down)

    sh = _run_shared(x, shared_gate_up, shared_down)
    a, b = _get_sc_gather()(rs, p0.reshape(T), p1.reshape(T))

    out = _run_combine(sh, a, b, w1, w2)
    return out.reshape(orig_shape)


# speedup vs baseline: 1.3561x; 1.0141x over previous
import functools

import jax
import jax.numpy as jnp
from jax import lax
from jax.experimental import pallas as pl
from jax.experimental.pallas import tpu as pltpu
from jax.experimental.pallas import tpu_sc as plsc

T = 2048
D = 1024
E = 8
FF = 1024
SF = 1024
TK = 2 * T
BLK = 512
NBP = TK // BLK + E - 1
NSTEPS = NBP
TKP = NBP * BLK
BT2 = 512

_F32 = jnp.float32
_BF16 = jnp.bfloat16
_I32 = jnp.int32


def _router_body(x_ref, gw_ref, bias_ref, pos_ref, p0_ref, p1_ref,
                 w1_ref, w2_ref, meta_ref):
    x = x_ref[...]
    logits = lax.dot_general(
        x, gw_ref[...], (((1,), (1,)), ((), ())),
        preferred_element_type=_F32,
        precision=lax.Precision.DEFAULT,
    )
    scores = jax.nn.sigmoid(logits)
    sfc = scores + bias_ref[...]

    lane = lax.broadcasted_iota(_I32, (T, E), 1)
    big = _F32(1e30)

    m1 = jnp.max(sfc, axis=1, keepdims=True)
    i1 = jnp.min(jnp.where(sfc >= m1, lane, E), axis=1, keepdims=True)
    oh1 = lane == i1
    sfc2 = jnp.where(oh1, -big, sfc)
    m2 = jnp.max(sfc2, axis=1, keepdims=True)
    i2 = jnp.min(jnp.where(sfc2 >= m2, lane, E), axis=1, keepdims=True)
    oh2 = lane == i2

    w1 = jnp.sum(jnp.where(oh1, scores, 0.0), axis=1, keepdims=True)
    w2 = jnp.sum(jnp.where(oh2, scores, 0.0), axis=1, keepdims=True)
    norm = w1 + w2 + 1e-20
    w1 = w1 / norm
    w2 = w2 / norm

    oh_f = oh1.astype(_F32) + oh2.astype(_F32)
    oh_b = oh_f.astype(_BF16)

    CH = 128
    tri = (lax.broadcasted_iota(_I32, (CH, CH), 0)
           > lax.broadcasted_iota(_I32, (CH, CH), 1)).astype(_BF16)
    chunks = []
    running = jnp.zeros((1, E), _F32)
    for c in range(T // CH):
        blk = oh_b[c * CH:(c + 1) * CH, :]
        within = lax.dot_general(
            tri, blk, (((1,), (0,)), ((), ())), preferred_element_type=_F32)
        chunks.append(within + running)
        running = running + jnp.sum(blk.astype(_F32), axis=0, keepdims=True)
    cexcl = jnp.concatenate(chunks, axis=0)
    counts = running

    cb = jnp.floor((counts + (BLK - 1.0)) * (1.0 / BLK))
    cb_b = jnp.broadcast_to(cb * BLK, (T, E))
    off1 = jnp.sum(jnp.where(lane < i1, cb_b, 0.0), axis=1, keepdims=True)
    off2 = jnp.sum(jnp.where(lane < i2, cb_b, 0.0), axis=1, keepdims=True)
    rank1 = jnp.sum(jnp.where(oh1, cexcl, 0.0), axis=1, keepdims=True)
    rank2 = jnp.sum(jnp.where(oh2, cexcl, 0.0), axis=1, keepdims=True)
    pos1 = (off1 + rank1).astype(_I32)
    pos2 = (off2 + rank2).astype(_I32)

    pos_ref[...] = jnp.concatenate([pos1, pos2], axis=1)
    p0_ref[...] = pos1
    p1_ref[...] = pos2
    w1_ref[...] = w1
    w2_ref[...] = w2

    ones_col = jnp.ones((T, 1), _BF16)
    counts_col = lax.dot_general(
        oh_b, ones_col, (((0,), (0,)), ((), ())),
        preferred_element_type=_F32)
    ltri = (lax.broadcasted_iota(_I32, (E, E), 1)
            < lax.broadcasted_iota(_I32, (E, E), 0)).astype(_BF16)
    cb_col = jnp.floor((counts_col + (BLK - 1.0)) * (1.0 / BLK))
    start = lax.dot_general(
        ltri, cb_col.astype(_BF16), (((1,), (0,)), ((), ())),
        preferred_element_type=_F32).astype(_I32)
    total = jnp.sum(cb_col).astype(_I32)

    svec = lax.broadcasted_iota(_I32, (E, 128), 1)
    s_eff = jnp.minimum(svec, total - 1)
    start_b2 = jnp.broadcast_to(start, (E, 128))
    e_of_s = jnp.sum((start_b2 <= s_eff).astype(_I32), axis=0,
                     keepdims=True) - 1
    blk_s = s_eff[0:1, :]
    valid = (svec[0:1, :] < total).astype(_I32)

    meta_ref[...] = jnp.concatenate(
        [e_of_s, blk_s, valid, jnp.zeros((5, 128), _I32)], axis=0)


def _run_router(x, gate_w, bias2):
    return pl.pallas_call(
        _router_body,
        out_shape=(
            jax.ShapeDtypeStruct((T, 2), _I32),
            jax.ShapeDtypeStruct((T, 1), _I32),
            jax.ShapeDtypeStruct((T, 1), _I32),
            jax.ShapeDtypeStruct((T, 1), _F32),
            jax.ShapeDtypeStruct((T, 1), _F32),
            jax.ShapeDtypeStruct((8, 128), _I32),
        ),
    )(x, gate_w, bias2)


def _make_sc_dispatch():
    mesh = plsc.VectorSubcoreMesh(core_axis_name="c", subcore_axis_name="s")
    NW = 32
    CHUNK = TK // NW
    SUB = CHUNK // 2

    @functools.partial(
        pl.kernel, mesh=mesh,
        out_type=jax.ShapeDtypeStruct((TKP, D), _F32),
        scratch_types=[
            pltpu.VMEM((SUB,), _I32),
            pltpu.VMEM((SUB,), _I32),
            pltpu.VMEM((SUB, D), _F32),
            pltpu.SemaphoreType.DMA,
        ],
    )
    def sc_dispatch(x_hbm, pos_hbm, xs_hbm, pos_v, tok_v, rows_v, sem):
        wid = lax.axis_index("s") * 2 + lax.axis_index("c")
        base = wid * CHUNK
        for sub in range(2):
            sbase = base + sub * SUB
            pltpu.sync_copy(pos_hbm.at[pl.ds(sbase, SUB)], pos_v)
            for j in range(SUB // 16):
                t16 = lax.shift_right_logical(
                    lax.iota(_I32, 16), 1) + ((sbase + 16 * j) // 2)
                tok_v[pl.ds(16 * j, 16)] = t16
            pltpu.async_copy(x_hbm.at[tok_v], rows_v, sem).wait()
            pltpu.async_copy(rows_v, xs_hbm.at[pos_v], sem).wait()

    return sc_dispatch


_SC_CACHE = {}


def _get_sc_dispatch():
    if "dispatch" not in _SC_CACHE:
        _SC_CACHE["dispatch"] = _make_sc_dispatch()
    return _SC_CACHE["dispatch"]


def _grouped_body(meta_ref, xs_ref, wgu_ref, wdn_ref, rs_ref):
    s = pl.program_id(0)
    valid = meta_ref[2, s] == 1

    @pl.when(valid)
    def _():
        xb = xs_ref[...].astype(_BF16)
        wgu = wgu_ref[0].astype(_BF16)
        gu = lax.dot_general(xb, wgu, (((1,), (1,)), ((), ())),
                             preferred_element_type=_F32)
        g = gu[:, :FF]
        u = gu[:, FF:]
        h = (g * jax.nn.sigmoid(g) * u).astype(_BF16)
        wdn = wdn_ref[0].astype(_BF16)
        eo = lax.dot_general(h, wdn, (((1,), (1,)), ((), ())),
                             preferred_element_type=_F32)
        rs_ref[...] = eo


def _run_grouped(meta, xs, w_gate_up, w_down):
    grid_spec = pltpu.PrefetchScalarGridSpec(
        num_scalar_prefetch=1,
        grid=(NSTEPS,),
        in_specs=[
            pl.BlockSpec((BLK, D), lambda s, m: (m[1, s], 0)),
            pl.BlockSpec((1, 2 * FF, D), lambda s, m: (m[0, s], 0, 0)),
            pl.BlockSpec((1, D, FF), lambda s, m: (m[0, s], 0, 0)),
        ],
        out_specs=pl.BlockSpec((BLK, D), lambda s, m: (m[1, s], 0)),
    )
    return pl.pallas_call(
        _grouped_body,
        grid_spec=grid_spec,
        out_shape=jax.ShapeDtypeStruct((TKP, D), _F32),
    )(meta, xs, w_gate_up, w_down)


def _make_sc_gather():
    mesh = plsc.VectorSubcoreMesh(core_axis_name="c", subcore_axis_name="s")
    NW = 32
    TPW = T // NW

    @functools.partial(
        pl.kernel, mesh=mesh,
        out_type=(
            jax.ShapeDtypeStruct((T, D), _F32),
            jax.ShapeDtypeStruct((T, D), _F32),
        ),
        scratch_types=[
            pltpu.VMEM((TPW,), _I32),
            pltpu.VMEM((TPW, D), _F32),
            pltpu.SemaphoreType.DMA,
        ],
    )
    def sc_gather(rs_hbm, p0_hbm, p1_hbm, a_hbm, b_hbm, idx_v, rows_v, sem):
        wid = lax.axis_index("s") * 2 + lax.axis_index("c")
        base = wid * TPW
        pltpu.sync_copy(p0_hbm.at[pl.ds(base, TPW)], idx_v)
        pltpu.async_copy(rs_hbm.at[idx_v], rows_v, sem).wait()
        pltpu.sync_copy(rows_v, a_hbm.at[pl.ds(base, TPW)])
        pltpu.sync_copy(p1_hbm.at[pl.ds(base, TPW)], idx_v)
        pltpu.async_copy(rs_hbm.at[idx_v], rows_v, sem).wait()
        pltpu.sync_copy(rows_v, b_hbm.at[pl.ds(base, TPW)])

    return sc_gather


def _get_sc_gather():
    if "gather" not in _SC_CACHE:
        _SC_CACHE["gather"] = _make_sc_gather()
    return _SC_CACHE["gather"]


def _shared_body(x_ref, sgu_ref, sdn_ref, out_ref, sgu_c, sdn_c):
    @pl.when(pl.program_id(0) == 0)
    def _():
        sgu_c[...] = sgu_ref[...].astype(_BF16)
        sdn_c[...] = sdn_ref[...].astype(_BF16)

    xb = x_ref[...].astype(_BF16)
    sgu = lax.dot_general(xb, sgu_c[...], (((1,), (1,)), ((), ())),
                          preferred_element_type=_F32)
    sg = sgu[:, :SF]
    su = sgu[:, SF:]
    sh = (sg * jax.nn.sigmoid(sg) * su).astype(_BF16)
    out_ref[...] = lax.dot_general(sh, sdn_c[...], (((1,), (1,)), ((), ())),
                                   preferred_element_type=_F32)


def _run_shared(x, shared_gate_up, shared_down):
    return pl.pallas_call(
        _shared_body,
        grid=(T // BT2,),
        in_specs=[
            pl.BlockSpec((BT2, D), lambda i: (i, 0)),
            pl.BlockSpec((2 * SF, D), lambda i: (0, 0)),
            pl.BlockSpec((D, SF), lambda i: (0, 0)),
        ],
        out_specs=pl.BlockSpec((BT2, D), lambda i: (i, 0)),
        out_shape=jax.ShapeDtypeStruct((T, D), _F32),
        scratch_shapes=[
            pltpu.VMEM((2 * SF, D), _BF16),
            pltpu.VMEM((D, SF), _BF16),
        ],
    )(x, shared_gate_up, shared_down)


def _combine_body(sh_ref, a_ref, b_ref, w1_ref, w2_ref, out_ref):
    out_ref[...] = (sh_ref[...] + w1_ref[...] * a_ref[...]
                    + w2_ref[...] * b_ref[...])


def _run_combine(sh, a, b, w1, w2):
    return pl.pallas_call(
        _combine_body,
        grid=(T // BT2,),
        in_specs=[
            pl.BlockSpec((BT2, D), lambda i: (i, 0)),
            pl.BlockSpec((BT2, D), lambda i: (i, 0)),
            pl.BlockSpec((BT2, D), lambda i: (i, 0)),
            pl.BlockSpec((BT2, 1), lambda i: (i, 0)),
            pl.BlockSpec((BT2, 1), lambda i: (i, 0)),
        ],
        out_specs=pl.BlockSpec((BT2, D), lambda i: (i, 0)),
        out_shape=jax.ShapeDtypeStruct((T, D), _F32),
    )(sh, a, b, w1, w2)


def kernel(hidden_states, gate_w, expert_bias, w_gate_up, w_down,
           shared_gate_up, shared_down):
    orig_shape = hidden_states.shape
    x = hidden_states.reshape(-1, orig_shape[-1])
    bias2 = expert_bias.reshape(1, E)

    pos2, p0, p1, w1, w2, meta = _run_router(x, gate_w, bias2)
    pos_flat = pos2.reshape(TK)

    xs = _get_sc_dispatch()(x, pos_flat)

    rs = _run_grouped(meta, xs, w_gate_up, w_down)

    sh = _run_shared(x, shared_gate_up, shared_down)
    a, b = _get_sc_gather()(rs, p0.reshape(T), p1.reshape(T))

    out = _run_combine(sh, a, b, w1, w2)
    return out.reshape(orig_shape)
